# initial kernel scaffold (unmeasured)
import jax
import jax.numpy as jnp
from jax import lax
from jax.experimental import pallas as pl
from jax.experimental.pallas import tpu as pltpu

N_DEV = 8


def kernel(x, w_mat):
    m, k_local = x.shape
    _, n = w_mat.shape
    mc = m // N_DEV
    n_hops = 2 * (N_DEV - 1)

    def body(x_ref, w_ref, out_ref, comm_ref, stage_ref,
             send_sems, recv_sems, credit_sem, copy_sems):
        d = lax.axis_index("i")
        left = (d + N_DEV - 1) % N_DEV
        right = (d + 1) % N_DEV

        barrier_sem = pltpu.get_barrier_semaphore()
        for nbr in (left, right):
            pl.semaphore_signal(
                barrier_sem, inc=1,
                device_id=(nbr,), device_id_type=pl.DeviceIdType.MESH,
            )
        pl.semaphore_wait(barrier_sem, 2)

        def partial_chunk(c):
            xc = x_ref[pl.ds(c * mc, mc), :]
            return jnp.dot(xc, w_ref[...], preferred_element_type=jnp.float32)

        comm_ref[0, :, :] = partial_chunk(d).astype(jnp.bfloat16)

        copies = []

        def store_chunk(idx, value):
            k = len(copies)
            cs = k % 2
            if k >= 2:
                copies[k - 2].wait()
            stage_ref[cs, :, :] = value.astype(jnp.float32)
            cp = pltpu.make_async_copy(
                stage_ref.at[cs],
                out_ref.at[pl.ds(idx * mc, mc), :],
                copy_sems.at[cs],
            )
            cp.start()
            copies.append(cp)

        for h in range(n_hops):
            s_slot = h % 2
            r_slot = (h + 1) % 2
            rdma = pltpu.make_async_remote_copy(
                src_ref=comm_ref.at[s_slot],
                dst_ref=comm_ref.at[r_slot],
                send_sem=send_sems.at[s_slot],
                recv_sem=recv_sems.at[r_slot],
                device_id=(right,),
                device_id_type=pl.DeviceIdType.MESH,
            )
            if h >= 1:
                pl.semaphore_wait(credit_sem, 1)
            rdma.start()
            if h < N_DEV - 1:
                c_recv = (d + 3 * N_DEV - 1 - h) % N_DEV
                p = partial_chunk(c_recv)
            rdma.wait()
            if h <= n_hops - 2:
                pl.semaphore_signal(
                    credit_sem, inc=1,
                    device_id=(left,), device_id_type=pl.DeviceIdType.MESH,
                )
            if h < N_DEV - 1:
                acc = comm_ref[r_slot, :, :].astype(jnp.float32) + p
                comm_ref[r_slot, :, :] = acc.astype(jnp.bfloat16)
                if h == N_DEV - 2:
                    store_chunk((d + 1) % N_DEV, comm_ref[r_slot, :, :])
            else:
                idx = (d + n_hops - h) % N_DEV
                store_chunk(idx, comm_ref[r_slot, :, :])

        copies[-2].wait()
        copies[-1].wait()

    out_shape = jax.ShapeDtypeStruct((m, n), jnp.float32)
    return pl.pallas_call(
        body,
        out_shape=out_shape,
        in_specs=[
            pl.BlockSpec(memory_space=pltpu.VMEM),
            pl.BlockSpec(memory_space=pltpu.VMEM),
        ],
        out_specs=pl.BlockSpec(memory_space=pltpu.ANY),
        scratch_shapes=[
            pltpu.VMEM((2, mc, n), jnp.bfloat16),
            pltpu.VMEM((2, mc, n), jnp.float32),
            pltpu.SemaphoreType.DMA((2,)),
            pltpu.SemaphoreType.DMA((2,)),
            pltpu.SemaphoreType.REGULAR,
            pltpu.SemaphoreType.DMA((2,)),
        ],
        compiler_params=pltpu.CompilerParams(collective_id=0),
    )(x, w_mat)


# baseline (device time: 1399670 ns/iter reference)
import jax
import jax.numpy as jnp
from jax import lax
from jax.experimental import pallas as pl
from jax.experimental.pallas import tpu as pltpu

N_DEV = 8


def kernel(x, w_mat):
    m, k_local = x.shape
    _, n = w_mat.shape
    mc = m // N_DEV
    n_hops = 2 * (N_DEV - 1)

    def body(x_ref, w_ref, out_ref, comm_ref, send_sems, recv_sems,
             credit_sem, copy_sems):
        d = lax.axis_index("i")
        left = (d + N_DEV - 1) % N_DEV
        right = (d + 1) % N_DEV

        barrier_sem = pltpu.get_barrier_semaphore()
        for nbr in (left, right):
            pl.semaphore_signal(
                barrier_sem, inc=1,
                device_id=(nbr,), device_id_type=pl.DeviceIdType.MESH,
            )
        pl.semaphore_wait(barrier_sem, 2)

        def partial_chunk(c):
            xc = x_ref[pl.ds(c * mc, mc), :]
            p32 = jnp.dot(xc, w_ref[...], preferred_element_type=jnp.float32)
            return p32.astype(jnp.bfloat16)

        def store_chunk(idx, slot, cs):
            cp = pltpu.make_async_copy(
                comm_ref.at[slot],
                out_ref.at[pl.ds(idx * mc, mc), :],
                copy_sems.at[cs],
            )
            cp.start()
            cp.wait()

        comm_ref[0, :, :] = partial_chunk(d)

        for h in range(n_hops):
            s_slot = h % 2
            r_slot = (h + 1) % 2
            rdma = pltpu.make_async_remote_copy(
                src_ref=comm_ref.at[s_slot],
                dst_ref=comm_ref.at[r_slot],
                send_sem=send_sems.at[s_slot],
                recv_sem=recv_sems.at[r_slot],
                device_id=(right,),
                device_id_type=pl.DeviceIdType.MESH,
            )
            if h >= 1:
                pl.semaphore_wait(credit_sem, 1)
            rdma.start()
            if h < N_DEV - 1:
                c_recv = (d + 3 * N_DEV - 1 - h) % N_DEV
                p = partial_chunk(c_recv)
            rdma.wait()
            if h < N_DEV - 1:
                comm_ref[r_slot, :, :] = comm_ref[r_slot, :, :] + p
                if h == N_DEV - 2:
                    store_chunk((d + 1) % N_DEV, r_slot, 0)
            else:
                idx = (d + 2 * N_DEV - (h - (N_DEV - 1))) % N_DEV
                store_chunk(idx, r_slot, (h - N_DEV + 2) % 2)
            if h <= n_hops - 2:
                pl.semaphore_signal(
                    credit_sem, inc=1,
                    device_id=(left,), device_id_type=pl.DeviceIdType.MESH,
                )

    out_shape = jax.ShapeDtypeStruct((m, n), jnp.bfloat16)
    return pl.pallas_call(
        body,
        out_shape=out_shape,
        in_specs=[
            pl.BlockSpec(memory_space=pltpu.MemorySpace.VMEM),
            pl.BlockSpec(memory_space=pltpu.MemorySpace.VMEM),
        ],
        out_specs=pl.BlockSpec(memory_space=pl.ANY),
        scratch_shapes=[
            pltpu.VMEM((2, mc, n), jnp.bfloat16),
            pltpu.SemaphoreType.DMA((2,)),
            pltpu.SemaphoreType.DMA((2,)),
            pltpu.SemaphoreType.REGULAR,
            pltpu.SemaphoreType.DMA((2,)),
        ],
        compiler_params=pltpu.CompilerParams(collective_id=0),
    )(x.astype(jnp.bfloat16), w_mat.astype(jnp.bfloat16))


# device time: 748209 ns/iter; 1.8707x vs baseline; 1.8707x over previous
import jax
import jax.numpy as jnp
from jax import lax
from jax.experimental import pallas as pl
from jax.experimental.pallas import tpu as pltpu

N_DEV = 8


def kernel(x, w_mat):
    m, k_local = x.shape
    _, n = w_mat.shape
    mc = m // N_DEV
    n2 = n // 2
    n_hops = 2 * (N_DEV - 1)

    def body(x_ref, w_ref, out_ref, comm_cw, comm_ccw,
             send_cw, recv_cw, send_ccw, recv_ccw,
             credit_cw, credit_ccw, copy_sems):
        d = lax.axis_index("i")
        left = (d + N_DEV - 1) % N_DEV
        right = (d + 1) % N_DEV

        barrier_sem = pltpu.get_barrier_semaphore()
        for nbr in (left, right):
            pl.semaphore_signal(
                barrier_sem, inc=1,
                device_id=(nbr,), device_id_type=pl.DeviceIdType.MESH,
            )
        pl.semaphore_wait(barrier_sem, 2)

        def partial_half(c, half):
            xc = x_ref[pl.ds(c * mc, mc), :]
            wh = w_ref[:, half * n2:(half + 1) * n2]
            p32 = jnp.dot(xc, wh, preferred_element_type=jnp.float32)
            return p32.astype(jnp.bfloat16)

        comm_cw[0, :, :] = partial_half(d, 0)
        comm_ccw[0, :, :] = partial_half(d, 1)

        pending = []

        def store_chunk(ref, slot, idx, half, cs):
            cp = pltpu.make_async_copy(
                ref.at[slot],
                out_ref.at[pl.ds(idx * mc, mc),
                           pl.ds(half * n2, n2)],
                copy_sems.at[half, cs],
            )
            cp.start()
            pending.append(cp)

        for h in range(n_hops):
            s_slot = h % 2
            r_slot = (h + 1) % 2
            rdma_cw = pltpu.make_async_remote_copy(
                src_ref=comm_cw.at[s_slot],
                dst_ref=comm_cw.at[r_slot],
                send_sem=send_cw.at[s_slot],
                recv_sem=recv_cw.at[r_slot],
                device_id=(right,),
                device_id_type=pl.DeviceIdType.MESH,
            )
            rdma_ccw = pltpu.make_async_remote_copy(
                src_ref=comm_ccw.at[s_slot],
                dst_ref=comm_ccw.at[r_slot],
                send_sem=send_ccw.at[s_slot],
                recv_sem=recv_ccw.at[r_slot],
                device_id=(left,),
                device_id_type=pl.DeviceIdType.MESH,
            )
            if h >= 1:
                pl.semaphore_wait(credit_cw, 1)
                pl.semaphore_wait(credit_ccw, 1)
            rdma_cw.start()
            rdma_ccw.start()
            if h < N_DEV - 1:
                c_cw = (d + 3 * N_DEV - 1 - h) % N_DEV
                c_ccw = (d + 1 + h) % N_DEV
                p_cw = partial_half(c_cw, 0)
                p_ccw = partial_half(c_ccw, 1)
            rdma_cw.wait()
            rdma_ccw.wait()
            if h < N_DEV - 1:
                comm_cw[r_slot, :, :] = comm_cw[r_slot, :, :] + p_cw
                comm_ccw[r_slot, :, :] = comm_ccw[r_slot, :, :] + p_ccw
                if h == N_DEV - 2:
                    store_chunk(comm_cw, r_slot, (d + 1) % N_DEV, 0, h % 2)
                    store_chunk(comm_ccw, r_slot, (d + N_DEV - 1) % N_DEV,
                                1, h % 2)
            else:
                s_ag = h - (N_DEV - 1)
                store_chunk(comm_cw, r_slot,
                            (d + 2 * N_DEV - s_ag) % N_DEV, 0, h % 2)
                store_chunk(comm_ccw, r_slot, (d + s_ag) % N_DEV, 1, h % 2)
            while len(pending) > 2:
                pending.pop(0).wait()
            if h <= n_hops - 2:
                pl.semaphore_signal(
                    credit_cw, inc=1,
                    device_id=(left,), device_id_type=pl.DeviceIdType.MESH,
                )
                pl.semaphore_signal(
                    credit_ccw, inc=1,
                    device_id=(right,), device_id_type=pl.DeviceIdType.MESH,
                )
        for cp in pending:
            cp.wait()

    out_shape = jax.ShapeDtypeStruct((m, n), jnp.bfloat16)
    return pl.pallas_call(
        body,
        out_shape=out_shape,
        in_specs=[
            pl.BlockSpec(memory_space=pltpu.MemorySpace.VMEM),
            pl.BlockSpec(memory_space=pltpu.MemorySpace.VMEM),
        ],
        out_specs=pl.BlockSpec(memory_space=pl.ANY),
        scratch_shapes=[
            pltpu.VMEM((2, mc, n2), jnp.bfloat16),
            pltpu.VMEM((2, mc, n2), jnp.bfloat16),
            pltpu.SemaphoreType.DMA((2,)),
            pltpu.SemaphoreType.DMA((2,)),
            pltpu.SemaphoreType.DMA((2,)),
            pltpu.SemaphoreType.DMA((2,)),
            pltpu.SemaphoreType.REGULAR,
            pltpu.SemaphoreType.REGULAR,
            pltpu.SemaphoreType.DMA((2, 2)),
        ],
        compiler_params=pltpu.CompilerParams(collective_id=0),
    )(x.astype(jnp.bfloat16), w_mat.astype(jnp.bfloat16))


# device time: 695840 ns/iter; 2.0115x vs baseline; 1.0753x over previous
import jax
import jax.numpy as jnp
from jax import lax
from jax.experimental import pallas as pl
from jax.experimental.pallas import tpu as pltpu

N_DEV = 8
N_HOPS = 2 * (N_DEV - 1)


def kernel(x, w_mat):
    m, k_local = x.shape
    _, n = w_mat.shape
    mc = m // N_DEV
    mc2 = mc // 2
    n2 = n // 2

    def body(x_ref, w_ref, out_ref,
             comm00, comm01, comm10, comm11,
             send_sems, recv_sems, credit_sems, copy_sems):
        d = lax.axis_index("i")
        left = (d + N_DEV - 1) % N_DEV
        right = (d + 1) % N_DEV
        comm = {(0, 0): comm00, (0, 1): comm01,
                (1, 0): comm10, (1, 1): comm11}
        send_to = {0: right, 1: left}
        recv_from = {0: left, 1: right}

        def partial_half(c, half):
            xc = x_ref[pl.ds(c * mc, mc), :]
            wh = w_ref[:, half * n2:(half + 1) * n2]
            p32 = jnp.dot(xc, wh, preferred_element_type=jnp.float32)
            return p32.astype(jnp.bfloat16)

        def rs_chunk(di, h):
            if di == 0:
                return (d + 3 * N_DEV - 1 - h) % N_DEV
            return (d + 1 + h) % N_DEV

        def desc(di, li, src_slot, dst_slot, target):
            return pltpu.make_async_remote_copy(
                src_ref=comm[(di, li)].at[src_slot],
                dst_ref=comm[(di, li)].at[dst_slot],
                send_sem=send_sems.at[di, li, src_slot],
                recv_sem=recv_sems.at[di, li, dst_slot],
                device_id=(target,),
                device_id_type=pl.DeviceIdType.MESH,
            )

        def send_desc(di, li, h):
            return desc(di, li, h % 2, (h + 1) % 2, send_to[di])

        def recv_desc(di, li, h):
            return desc(di, li, h % 2, (h + 1) % 2, recv_from[di])

        def copy_desc(di, li, h):
            if h == N_DEV - 2:
                idx = (d + 1) % N_DEV if di == 0 else (d + N_DEV - 1) % N_DEV
            else:
                s_ag = h - (N_DEV - 1)
                idx = ((d + 2 * N_DEV - s_ag) if di == 0 else (d + s_ag)) % N_DEV
            return pltpu.make_async_copy(
                comm[(di, li)].at[(h + 1) % 2],
                out_ref.at[pl.ds(idx * mc + li * mc2, mc2),
                           pl.ds(di * n2, n2)],
                copy_sems.at[di, li, h % 2],
            )

        p = [partial_half(d, 0), partial_half(d, 1)]
        for di in (0, 1):
            for li in (0, 1):
                comm[(di, li)][0, :, :] = p[di][li * mc2:(li + 1) * mc2, :]

        barrier_sem = pltpu.get_barrier_semaphore()
        for nbr in (left, right):
            pl.semaphore_signal(
                barrier_sem, inc=1,
                device_id=(nbr,), device_id_type=pl.DeviceIdType.MESH,
            )
        pl.semaphore_wait(barrier_sem, 2)

        for li in (0, 1):
            for di in (0, 1):
                send_desc(di, li, 0).start()
        p = [partial_half(rs_chunk(0, 0), 0), partial_half(rs_chunk(1, 0), 1)]

        for h in range(N_HOPS):
            for li in (0, 1):
                for di in (0, 1):
                    recv_desc(di, li, h).wait_recv()
                    send_desc(di, li, h).wait_send()
                    if h >= N_DEV - 1:
                        copy_desc(di, li, h - 1).wait()
                    if h <= N_HOPS - 2:
                        pl.semaphore_signal(
                            credit_sems.at[di, li], inc=1,
                            device_id=(recv_from[di],),
                            device_id_type=pl.DeviceIdType.MESH,
                        )
                    r_slot = (h + 1) % 2
                    if h < N_DEV - 1:
                        buf = comm[(di, li)]
                        buf[r_slot, :, :] = (
                            buf[r_slot, :, :]
                            + p[di][li * mc2:(li + 1) * mc2, :]
                        )
                    if h >= N_DEV - 2:
                        copy_desc(di, li, h).start()
                    if h <= N_HOPS - 2:
                        pl.semaphore_wait(credit_sems.at[di, li], 1)
                        send_desc(di, li, h + 1).start()
            if h + 1 < N_DEV - 1:
                p = [partial_half(rs_chunk(0, h + 1), 0),
                     partial_half(rs_chunk(1, h + 1), 1)]

        for li in (0, 1):
            for di in (0, 1):
                copy_desc(di, li, N_HOPS - 1).wait()

    out_shape = jax.ShapeDtypeStruct((m, n), jnp.bfloat16)
    return pl.pallas_call(
        body,
        out_shape=out_shape,
        in_specs=[
            pl.BlockSpec(memory_space=pltpu.MemorySpace.VMEM),
            pl.BlockSpec(memory_space=pltpu.MemorySpace.VMEM),
        ],
        out_specs=pl.BlockSpec(memory_space=pl.ANY),
        scratch_shapes=[
            pltpu.VMEM((2, mc2, n2), jnp.bfloat16),
            pltpu.VMEM((2, mc2, n2), jnp.bfloat16),
            pltpu.VMEM((2, mc2, n2), jnp.bfloat16),
            pltpu.VMEM((2, mc2, n2), jnp.bfloat16),
            pltpu.SemaphoreType.DMA((2, 2, 2)),
            pltpu.SemaphoreType.DMA((2, 2, 2)),
            pltpu.SemaphoreType.REGULAR((2, 2)),
            pltpu.SemaphoreType.DMA((2, 2, 2)),
        ],
        compiler_params=pltpu.CompilerParams(collective_id=0),
    )(x.astype(jnp.bfloat16), w_mat.astype(jnp.bfloat16))
